# baseline (speedup 1.0000x reference)
import jax
import jax.numpy as jnp
from jax import lax
from jax.experimental import pallas as pl
from jax.experimental.pallas import tpu as pltpu
import jax.experimental.pallas.tpu_sc as plsc

N = 262144
NC = 2
NS = 16
NW = NC * NS
GCHUNK = N // NW
DEST = 6 * 256 * 256
OWN = DEST // NW
SCHUNK = 16384
NSCHUNK = N // SCHUNK
TBL = 6 * 1000000
LCH = 32256
LTC = 999936

_mesh = plsc.VectorSubcoreMesh(
    core_axis_name="c", subcore_axis_name="s", num_cores=NC, num_subcores=NS
)
_sc_params = pltpu.CompilerParams(needs_layout_passes=False)


def _wid():
    return lax.axis_index("s") * NC + lax.axis_index("c")


def _prep_body(p3_hbm, p4_hbm, p5_hbm, p6_hbm,
               dest_hbm, lin_hbm,
               i3, i4, i5, i6, dst, lin):
    wid = _wid()
    base = wid * GCHUNK

    pltpu.sync_copy(p3_hbm.at[pl.ds(base, GCHUNK)], i3)
    pltpu.sync_copy(p4_hbm.at[pl.ds(base, GCHUNK)], i4)
    pltpu.sync_copy(p5_hbm.at[pl.ds(base, GCHUNK)], i5)
    pltpu.sync_copy(p6_hbm.at[pl.ds(base, GCHUNK)], i6)

    def body(j, carry):
        base_j = j * 128
        for k in range(8):
            sl = pl.ds(base_j + k * 16, 16)
            a3 = i3[sl]
            lin[sl] = a3 * 1000000 + i4[sl]
            dst[sl] = a3 * 65536 + i5[sl] * 256 + i6[sl]
        return carry

    lax.fori_loop(0, GCHUNK // 128, body, 0)
    pltpu.sync_copy(dst, dest_hbm.at[pl.ds(base, GCHUNK)])
    pltpu.sync_copy(lin, lin_hbm.at[pl.ds(base, GCHUNK)])


_prep_call = pl.kernel(
    _prep_body,
    out_type=(
        jax.ShapeDtypeStruct((N,), jnp.int32),
        jax.ShapeDtypeStruct((N,), jnp.int32),
    ),
    mesh=_mesh,
    scratch_types=[
        pltpu.VMEM((GCHUNK,), jnp.int32),
        pltpu.VMEM((GCHUNK,), jnp.int32),
        pltpu.VMEM((GCHUNK,), jnp.int32),
        pltpu.VMEM((GCHUNK,), jnp.int32),
        pltpu.VMEM((GCHUNK,), jnp.int32),
        pltpu.VMEM((GCHUNK,), jnp.int32),
    ],
    compiler_params=_sc_params,
)


def _gather_body(p2lin_hbm, lin_hbm, vals_hbm, linv, vals, sem):
    base = _wid() * GCHUNK
    pltpu.sync_copy(lin_hbm.at[pl.ds(base, GCHUNK)], linv)
    pltpu.async_copy(p2lin_hbm.at[linv], vals, sem).wait()
    pltpu.sync_copy(vals, vals_hbm.at[pl.ds(base, GCHUNK)])


_gather_call = pl.kernel(
    _gather_body,
    out_type=jax.ShapeDtypeStruct((N,), jnp.float32),
    mesh=_mesh,
    scratch_types=[
        pltpu.VMEM((GCHUNK,), jnp.int32),
        pltpu.VMEM((GCHUNK,), jnp.float32),
        pltpu.SemaphoreType.DMA,
    ],
    compiler_params=_sc_params,
)


def _scatter_body(dest_hbm, vals_hbm, p1_hbm, out_hbm, local,
                  dbuf0, dbuf1, vbuf0, vbuf1, semd, semv):
    wid = _wid()
    lo = wid * OWN
    pltpu.sync_copy(p1_hbm.at[pl.ds(lo, OWN)], local.at[pl.ds(0, OWN)])
    lo_v = jnp.full((16,), 0, jnp.int32) + lo
    lane = lax.broadcasted_iota(jnp.int32, (16,), 0)
    own_v = plsc.bitcast(lane + OWN, jnp.uint32)

    dbufs = (dbuf0, dbuf1)
    vbufs = (vbuf0, vbuf1)

    def start(c, b):
        pltpu.async_copy(dest_hbm.at[pl.ds(c * SCHUNK, SCHUNK)], dbufs[b], semd)
        pltpu.async_copy(vals_hbm.at[pl.ds(c * SCHUNK, SCHUNK)], vbufs[b], semv)

    def wait(c, b):
        pltpu.make_async_copy(dest_hbm.at[pl.ds(c * SCHUNK, SCHUNK)],
                              dbufs[b], semd).wait()
        pltpu.make_async_copy(vals_hbm.at[pl.ds(c * SCHUNK, SCHUNK)],
                              vbufs[b], semv).wait()

    start(0, 0)
    for c in range(NSCHUNK):
        b = c % 2
        wait(c, b)
        if c + 1 < NSCHUNK:
            start(c + 1, 1 - b)
        dbuf = dbufs[b]
        vbuf = vbufs[b]

        def inner(j, carry):
            base_j = j * 128
            us = []
            for k in range(8):
                sl = pl.ds(base_j + k * 16, 16)
                u = plsc.bitcast(dbuf[sl] - lo_v, jnp.uint32)
                us.append(jnp.minimum(u, own_v))
            vs = [vbuf[pl.ds(base_j + k * 16, 16)] for k in range(8)]
            for k in range(8):
                plsc.store_scatter(local, [plsc.bitcast(us[k], jnp.int32)],
                                   vs[k])
            return carry

        lax.fori_loop(0, SCHUNK // 128, inner, 0)

    pltpu.sync_copy(local.at[pl.ds(0, OWN)], out_hbm.at[pl.ds(lo, OWN)])


_scatter_call = pl.kernel(
    _scatter_body,
    out_type=jax.ShapeDtypeStruct((DEST,), jnp.float32),
    mesh=_mesh,
    scratch_types=[
        pltpu.VMEM((OWN + 16,), jnp.float32),
        pltpu.VMEM((SCHUNK,), jnp.int32),
        pltpu.VMEM((SCHUNK,), jnp.int32),
        pltpu.VMEM((SCHUNK,), jnp.float32),
        pltpu.VMEM((SCHUNK,), jnp.float32),
        pltpu.SemaphoreType.DMA,
        pltpu.SemaphoreType.DMA,
    ],
    compiler_params=_sc_params,
)


def _tc_body(ip_ref, p7_ref, v_ref, bmm_ref, pm6_ref):
    add = ip_ref[0] + p7_ref[0] * 0.975
    bmm_ref[0] = jnp.dot(v_ref[0], add, preferred_element_type=jnp.float32)
    pm6_ref[0] = add.T


_tc_call = pl.pallas_call(
    _tc_body,
    grid=(6,),
    in_specs=[
        pl.BlockSpec((1, 256, 256), lambda b: (b, 0, 0)),
        pl.BlockSpec((1, 256, 256), lambda b: (b, 0, 0)),
        pl.BlockSpec((1, 12, 256), lambda b: (b, 0, 0)),
    ],
    out_specs=[
        pl.BlockSpec((1, 12, 256), lambda b: (b, 0, 0)),
        pl.BlockSpec((1, 256, 256), lambda b: (b, 0, 0)),
    ],
    out_shape=[
        jax.ShapeDtypeStruct((6, 12, 256), jnp.float32),
        jax.ShapeDtypeStruct((6, 256, 256), jnp.float32),
    ],
)


@jax.jit
def kernel(primals_1, primals_2, primals_3, primals_4, primals_5, primals_6,
           primals_7, primals_8):
    p1f = primals_1.reshape(-1)
    p2lin = primals_2.reshape(-1)
    dest, lin = _prep_call(primals_3, primals_4, primals_5, primals_6)
    vals = _gather_call(p2lin, lin)
    ipf = _scatter_call(dest, vals, p1f)
    ip = ipf.reshape(6, 256, 256)
    view = jnp.transpose(primals_8, (1, 0, 2))
    bmm6, pm6 = _tc_call(ip, primals_7, view)
    view_3 = jnp.transpose(bmm6, (1, 0, 2))
    return (view_3, pm6)


# speedup vs baseline: 4.8771x; 4.8771x over previous
import jax
import jax.numpy as jnp
from jax import lax
from jax.experimental import pallas as pl
from jax.experimental.pallas import tpu as pltpu
import jax.experimental.pallas.tpu_sc as plsc

N = 262144
NC = 2
NS = 16
NW = NC * NS
GCHUNK = N // NW
DEST = 6 * 256 * 256
OWN2 = DEST // NS
SCHUNK = 8192
NSCHUNK = (N // 2) // SCHUNK
TBL = 6 * 1000000
LCH = 32256
LTC = 999936

_mesh = plsc.VectorSubcoreMesh(
    core_axis_name="c", subcore_axis_name="s", num_cores=NC, num_subcores=NS
)
_sc_params = pltpu.CompilerParams(needs_layout_passes=False)


def _wid():
    return lax.axis_index("s") * NC + lax.axis_index("c")


def _prep_body(p2_hbm, p3_hbm, p4_hbm, p5_hbm, p6_hbm,
               p2lin_hbm, dest_hbm, lin_hbm,
               i3, i4, i5, i6, dst, lin, b0, b1, tbuf, sem):
    wid = _wid()
    base = wid * GCHUNK
    bufs = (b0, b1)

    @pl.when(wid < 31)
    def _():
        for r in range(6):
            buf = bufs[r % 2]
            src = p2_hbm.at[r, pl.ds(wid * LCH, LCH)]
            out = p2lin_hbm.at[pl.ds(r * 1000000 + wid * LCH, LCH)]
            if r >= 2:
                prev = p2lin_hbm.at[pl.ds((r - 2) * 1000000 + wid * LCH, LCH)]
                pltpu.make_async_copy(buf, prev, sem).wait()
            pltpu.sync_copy(src, buf)
            pltpu.async_copy(buf, out, sem)
        for r in range(4, 6):
            out = p2lin_hbm.at[pl.ds(r * 1000000 + wid * LCH, LCH)]
            pltpu.make_async_copy(bufs[r % 2], out, sem).wait()

    @pl.when(wid == 31)
    def _():
        pltpu.sync_copy(p2_hbm.at[pl.ds(0, 6), pl.ds(LTC, 64)], tbuf)
        for r in range(6):
            pltpu.sync_copy(tbuf.at[r],
                            p2lin_hbm.at[pl.ds(r * 1000000 + LTC, 64)])

    pltpu.sync_copy(p3_hbm.at[pl.ds(base, GCHUNK)], i3)
    pltpu.sync_copy(p4_hbm.at[pl.ds(base, GCHUNK)], i4)
    pltpu.sync_copy(p5_hbm.at[pl.ds(base, GCHUNK)], i5)
    pltpu.sync_copy(p6_hbm.at[pl.ds(base, GCHUNK)], i6)

    def body(j, carry):
        base_j = j * 128
        for k in range(8):
            sl = pl.ds(base_j + k * 16, 16)
            a3 = i3[sl]
            lin[sl] = a3 * 1000000 + i4[sl]
            dst[sl] = a3 * 65536 + i5[sl] * 256 + i6[sl]
        return carry

    lax.fori_loop(0, GCHUNK // 128, body, 0)
    pltpu.sync_copy(dst, dest_hbm.at[pl.ds(base, GCHUNK)])
    pltpu.sync_copy(lin, lin_hbm.at[pl.ds(base, GCHUNK)])


_prep_call = pl.kernel(
    _prep_body,
    out_type=(
        jax.ShapeDtypeStruct((TBL,), jnp.float32),
        jax.ShapeDtypeStruct((N,), jnp.int32),
        jax.ShapeDtypeStruct((N,), jnp.int32),
    ),
    mesh=_mesh,
    scratch_types=[
        pltpu.VMEM((GCHUNK,), jnp.int32),
        pltpu.VMEM((GCHUNK,), jnp.int32),
        pltpu.VMEM((GCHUNK,), jnp.int32),
        pltpu.VMEM((GCHUNK,), jnp.int32),
        pltpu.VMEM((GCHUNK,), jnp.int32),
        pltpu.VMEM((GCHUNK,), jnp.int32),
        pltpu.VMEM((LCH,), jnp.float32),
        pltpu.VMEM((LCH,), jnp.float32),
        pltpu.VMEM((6, 64), jnp.float32),
        pltpu.SemaphoreType.DMA,
    ],
    compiler_params=_sc_params,
)


def _gather_body(p2lin_hbm, lin_hbm, vals_hbm, linv, vals, sem):
    base = _wid() * GCHUNK
    pltpu.sync_copy(lin_hbm.at[pl.ds(base, GCHUNK)], linv)
    pltpu.async_copy(p2lin_hbm.at[linv], vals, sem).wait()
    pltpu.sync_copy(vals, vals_hbm.at[pl.ds(base, GCHUNK)])


_gather_call = pl.kernel(
    _gather_body,
    out_type=jax.ShapeDtypeStruct((N,), jnp.float32),
    mesh=_mesh,
    scratch_types=[
        pltpu.VMEM((GCHUNK,), jnp.int32),
        pltpu.VMEM((GCHUNK,), jnp.float32),
        pltpu.SemaphoreType.DMA,
    ],
    compiler_params=_sc_params,
)


def _scatter_body(dest_hbm, vals_hbm, out0_hbm, out1_hbm, m0_hbm, m1_hbm,
                  local, lmask, dbuf0, dbuf1, vbuf0, vbuf1, semd, semv):
    cid = lax.axis_index("c")
    sid = lax.axis_index("s")
    lo = sid * OWN2
    hbase = cid * (N // 2)
    lo_v = jnp.full((16,), 0, jnp.int32) + lo
    lane = lax.broadcasted_iota(jnp.int32, (16,), 0)
    own_v = plsc.bitcast(lane + OWN2, jnp.uint32)
    ones = jnp.full((16,), 1.0, jnp.float32)
    zeros = jnp.full((16,), 0.0, jnp.float32)

    def zbody(j, carry):
        base_j = j * 128
        for k in range(8):
            lmask[pl.ds(base_j + k * 16, 16)] = zeros
        return carry

    lax.fori_loop(0, (OWN2 + 16) // 128, zbody, 0)

    dbufs = (dbuf0, dbuf1)
    vbufs = (vbuf0, vbuf1)

    def start(c, b):
        sl = pl.ds(hbase + c * SCHUNK, SCHUNK)
        pltpu.async_copy(dest_hbm.at[sl], dbufs[b], semd)
        pltpu.async_copy(vals_hbm.at[sl], vbufs[b], semv)

    def wait(c, b):
        sl = pl.ds(hbase + c * SCHUNK, SCHUNK)
        pltpu.make_async_copy(dest_hbm.at[sl], dbufs[b], semd).wait()
        pltpu.make_async_copy(vals_hbm.at[sl], vbufs[b], semv).wait()

    start(0, 0)
    for c in range(NSCHUNK):
        b = c % 2
        wait(c, b)
        if c + 1 < NSCHUNK:
            start(c + 1, 1 - b)
        dbuf = dbufs[b]
        vbuf = vbufs[b]

        def inner(j, carry):
            base_j = j * 128
            us = []
            for k in range(8):
                sl = pl.ds(base_j + k * 16, 16)
                u = plsc.bitcast(dbuf[sl] - lo_v, jnp.uint32)
                us.append(plsc.bitcast(jnp.minimum(u, own_v), jnp.int32))
            vs = [vbuf[pl.ds(base_j + k * 16, 16)] for k in range(8)]
            for k in range(8):
                plsc.store_scatter(local, [us[k]], vs[k])
                plsc.store_scatter(lmask, [us[k]], ones)
            return carry

        lax.fori_loop(0, SCHUNK // 128, inner, 0)

    @pl.when(cid == 0)
    def _():
        pltpu.sync_copy(local.at[pl.ds(0, OWN2)], out0_hbm.at[pl.ds(lo, OWN2)])
        pltpu.sync_copy(lmask.at[pl.ds(0, OWN2)], m0_hbm.at[pl.ds(lo, OWN2)])

    @pl.when(cid == 1)
    def _():
        pltpu.sync_copy(local.at[pl.ds(0, OWN2)], out1_hbm.at[pl.ds(lo, OWN2)])
        pltpu.sync_copy(lmask.at[pl.ds(0, OWN2)], m1_hbm.at[pl.ds(lo, OWN2)])


_scatter_call = pl.kernel(
    _scatter_body,
    out_type=(
        jax.ShapeDtypeStruct((DEST,), jnp.float32),
        jax.ShapeDtypeStruct((DEST,), jnp.float32),
        jax.ShapeDtypeStruct((DEST,), jnp.float32),
        jax.ShapeDtypeStruct((DEST,), jnp.float32),
    ),
    mesh=_mesh,
    scratch_types=[
        pltpu.VMEM((OWN2 + 16,), jnp.float32),
        pltpu.VMEM((OWN2 + 16,), jnp.float32),
        pltpu.VMEM((SCHUNK,), jnp.int32),
        pltpu.VMEM((SCHUNK,), jnp.int32),
        pltpu.VMEM((SCHUNK,), jnp.float32),
        pltpu.VMEM((SCHUNK,), jnp.float32),
        pltpu.SemaphoreType.DMA,
        pltpu.SemaphoreType.DMA,
    ],
    compiler_params=_sc_params,
)


def _tc_body(o0_ref, o1_ref, m0_ref, m1_ref, p1_ref, p7_ref, v_ref,
             bmm_ref, pm6_ref):
    ip = jnp.where(m1_ref[0] > 0.5, o1_ref[0],
                   jnp.where(m0_ref[0] > 0.5, o0_ref[0], p1_ref[0]))
    add = ip + p7_ref[0] * 0.975
    bmm_ref[0] = jnp.dot(v_ref[0], add, preferred_element_type=jnp.float32)
    pm6_ref[0] = add.T


_b3 = pl.BlockSpec((1, 256, 256), lambda b: (b, 0, 0))
_tc_call = pl.pallas_call(
    _tc_body,
    grid=(6,),
    in_specs=[
        _b3, _b3, _b3, _b3, _b3, _b3,
        pl.BlockSpec((1, 12, 256), lambda b: (b, 0, 0)),
    ],
    out_specs=[
        pl.BlockSpec((1, 12, 256), lambda b: (b, 0, 0)),
        _b3,
    ],
    out_shape=[
        jax.ShapeDtypeStruct((6, 12, 256), jnp.float32),
        jax.ShapeDtypeStruct((6, 256, 256), jnp.float32),
    ],
)


@jax.jit
def kernel(primals_1, primals_2, primals_3, primals_4, primals_5, primals_6,
           primals_7, primals_8):
    p2lin, dest, lin = _prep_call(primals_2, primals_3, primals_4,
                                  primals_5, primals_6)
    vals = _gather_call(p2lin, lin)
    o0f, o1f, m0f, m1f = _scatter_call(dest, vals)
    o0 = o0f.reshape(6, 256, 256)
    o1 = o1f.reshape(6, 256, 256)
    m0 = m0f.reshape(6, 256, 256)
    m1 = m1f.reshape(6, 256, 256)
    view = jnp.transpose(primals_8, (1, 0, 2))
    bmm6, pm6 = _tc_call(o0, o1, m0, m1, primals_1, primals_7, view)
    view_3 = jnp.transpose(bmm6, (1, 0, 2))
    return (view_3, pm6)


# speedup vs baseline: 4.8870x; 1.0020x over previous
import jax
import jax.numpy as jnp
from jax import lax
from jax.experimental import pallas as pl
from jax.experimental.pallas import tpu as pltpu
import jax.experimental.pallas.tpu_sc as plsc

N = 262144
NC = 2
NS = 16
NW = NC * NS
GCHUNK = N // NW
DEST = 6 * 256 * 256
NSEG = 4
NRNG = NW // NSEG
OWN = DEST // NRNG
SCHUNK = 4096
NSCHUNK = (N // NSEG) // SCHUNK
TBL = 6 * 1000000
LCH = 32256
LTC = 999936

_mesh = plsc.VectorSubcoreMesh(
    core_axis_name="c", subcore_axis_name="s", num_cores=NC, num_subcores=NS
)
_sc_params = pltpu.CompilerParams(needs_layout_passes=False)


def _wid():
    return lax.axis_index("s") * NC + lax.axis_index("c")


def _prep_body(p2_hbm, p3_hbm, p4_hbm, p5_hbm, p6_hbm,
               p2lin_hbm, dest_hbm, lin_hbm,
               i3, i4, i5, i6, dst, lin, b0, b1, tbuf, sem):
    wid = _wid()
    base = wid * GCHUNK
    bufs = (b0, b1)

    @pl.when(wid < 31)
    def _():
        for r in range(6):
            buf = bufs[r % 2]
            src = p2_hbm.at[r, pl.ds(wid * LCH, LCH)]
            out = p2lin_hbm.at[pl.ds(r * 1000000 + wid * LCH, LCH)]
            if r >= 2:
                prev = p2lin_hbm.at[pl.ds((r - 2) * 1000000 + wid * LCH, LCH)]
                pltpu.make_async_copy(buf, prev, sem).wait()
            pltpu.sync_copy(src, buf)
            pltpu.async_copy(buf, out, sem)
        for r in range(4, 6):
            out = p2lin_hbm.at[pl.ds(r * 1000000 + wid * LCH, LCH)]
            pltpu.make_async_copy(bufs[r % 2], out, sem).wait()

    @pl.when(wid == 31)
    def _():
        pltpu.sync_copy(p2_hbm.at[pl.ds(0, 6), pl.ds(LTC, 64)], tbuf)
        for r in range(6):
            pltpu.sync_copy(tbuf.at[r],
                            p2lin_hbm.at[pl.ds(r * 1000000 + LTC, 64)])

    pltpu.sync_copy(p3_hbm.at[pl.ds(base, GCHUNK)], i3)
    pltpu.sync_copy(p4_hbm.at[pl.ds(base, GCHUNK)], i4)
    pltpu.sync_copy(p5_hbm.at[pl.ds(base, GCHUNK)], i5)
    pltpu.sync_copy(p6_hbm.at[pl.ds(base, GCHUNK)], i6)

    def body(j, carry):
        base_j = j * 128
        for k in range(8):
            sl = pl.ds(base_j + k * 16, 16)
            a3 = i3[sl]
            lin[sl] = a3 * 1000000 + i4[sl]
            dst[sl] = a3 * 65536 + i5[sl] * 256 + i6[sl]
        return carry

    lax.fori_loop(0, GCHUNK // 128, body, 0)
    pltpu.sync_copy(dst, dest_hbm.at[pl.ds(base, GCHUNK)])
    pltpu.sync_copy(lin, lin_hbm.at[pl.ds(base, GCHUNK)])


_prep_call = pl.kernel(
    _prep_body,
    out_type=(
        jax.ShapeDtypeStruct((TBL,), jnp.float32),
        jax.ShapeDtypeStruct((N,), jnp.int32),
        jax.ShapeDtypeStruct((N,), jnp.int32),
    ),
    mesh=_mesh,
    scratch_types=[
        pltpu.VMEM((GCHUNK,), jnp.int32),
        pltpu.VMEM((GCHUNK,), jnp.int32),
        pltpu.VMEM((GCHUNK,), jnp.int32),
        pltpu.VMEM((GCHUNK,), jnp.int32),
        pltpu.VMEM((GCHUNK,), jnp.int32),
        pltpu.VMEM((GCHUNK,), jnp.int32),
        pltpu.VMEM((LCH,), jnp.float32),
        pltpu.VMEM((LCH,), jnp.float32),
        pltpu.VMEM((6, 64), jnp.float32),
        pltpu.SemaphoreType.DMA,
    ],
    compiler_params=_sc_params,
)


def _gather_body(p2lin_hbm, lin_hbm, vals_hbm, linv, vals, sem):
    base = _wid() * GCHUNK
    pltpu.sync_copy(lin_hbm.at[pl.ds(base, GCHUNK)], linv)
    pltpu.async_copy(p2lin_hbm.at[linv], vals, sem).wait()
    pltpu.sync_copy(vals, vals_hbm.at[pl.ds(base, GCHUNK)])


_gather_call = pl.kernel(
    _gather_body,
    out_type=jax.ShapeDtypeStruct((N,), jnp.float32),
    mesh=_mesh,
    scratch_types=[
        pltpu.VMEM((GCHUNK,), jnp.int32),
        pltpu.VMEM((GCHUNK,), jnp.float32),
        pltpu.SemaphoreType.DMA,
    ],
    compiler_params=_sc_params,
)


def _scatter_body(dest_hbm, vals_hbm,
                  out0_hbm, out1_hbm, out2_hbm, out3_hbm,
                  m0_hbm, m1_hbm, m2_hbm, m3_hbm,
                  local, lmask, dbuf0, dbuf1, vbuf0, vbuf1, semd, semv):
    cid = lax.axis_index("c")
    sid = lax.axis_index("s")
    seg = cid * 2 + sid // NRNG
    rid = sid % NRNG
    lo = rid * OWN
    hbase = seg * (N // NSEG)
    lo_v = jnp.full((16,), 0, jnp.int32) + lo
    lane = lax.broadcasted_iota(jnp.int32, (16,), 0)
    own_v = plsc.bitcast(lane + OWN, jnp.uint32)
    ones = jnp.full((16,), 1.0, jnp.float32)
    zeros = jnp.full((16,), 0.0, jnp.float32)

    def zbody(j, carry):
        base_j = j * 128
        for k in range(8):
            lmask[pl.ds(base_j + k * 16, 16)] = zeros
        return carry

    lax.fori_loop(0, OWN // 128, zbody, 0)

    dbufs = (dbuf0, dbuf1)
    vbufs = (vbuf0, vbuf1)

    def start(c, b):
        sl = pl.ds(hbase + c * SCHUNK, SCHUNK)
        pltpu.async_copy(dest_hbm.at[sl], dbufs[b], semd)
        pltpu.async_copy(vals_hbm.at[sl], vbufs[b], semv)

    def wait(c, b):
        sl = pl.ds(hbase + c * SCHUNK, SCHUNK)
        pltpu.make_async_copy(dest_hbm.at[sl], dbufs[b], semd).wait()
        pltpu.make_async_copy(vals_hbm.at[sl], vbufs[b], semv).wait()

    start(0, 0)
    for c in range(NSCHUNK):
        b = c % 2
        wait(c, b)
        if c + 1 < NSCHUNK:
            start(c + 1, 1 - b)
        dbuf = dbufs[b]
        vbuf = vbufs[b]

        def inner(j, carry):
            base_j = j * 128
            us = []
            for k in range(8):
                sl = pl.ds(base_j + k * 16, 16)
                u = plsc.bitcast(dbuf[sl] - lo_v, jnp.uint32)
                us.append(plsc.bitcast(jnp.minimum(u, own_v), jnp.int32))
            vs = [vbuf[pl.ds(base_j + k * 16, 16)] for k in range(8)]
            for k in range(8):
                plsc.store_scatter(local, [us[k]], vs[k])
                plsc.store_scatter(lmask, [us[k]], ones)
            return carry

        lax.fori_loop(0, SCHUNK // 128, inner, 0)

    outs_hbm = (out0_hbm, out1_hbm, out2_hbm, out3_hbm)
    ms_hbm = (m0_hbm, m1_hbm, m2_hbm, m3_hbm)
    for k in range(NSEG):
        @pl.when(seg == k)
        def _(k=k):
            pltpu.sync_copy(local.at[pl.ds(0, OWN)],
                            outs_hbm[k].at[pl.ds(lo, OWN)])
            pltpu.sync_copy(lmask.at[pl.ds(0, OWN)],
                            ms_hbm[k].at[pl.ds(lo, OWN)])


_scatter_call = pl.kernel(
    _scatter_body,
    out_type=tuple(
        jax.ShapeDtypeStruct((DEST,), jnp.float32) for _ in range(2 * NSEG)
    ),
    mesh=_mesh,
    scratch_types=[
        pltpu.VMEM((OWN + 16,), jnp.float32),
        pltpu.VMEM((OWN + 16,), jnp.float32),
        pltpu.VMEM((SCHUNK,), jnp.int32),
        pltpu.VMEM((SCHUNK,), jnp.int32),
        pltpu.VMEM((SCHUNK,), jnp.float32),
        pltpu.VMEM((SCHUNK,), jnp.float32),
        pltpu.SemaphoreType.DMA,
        pltpu.SemaphoreType.DMA,
    ],
    compiler_params=_sc_params,
)


def _tc_body(o0_ref, o1_ref, o2_ref, o3_ref,
             m0_ref, m1_ref, m2_ref, m3_ref, p1_ref, p7_ref, v_ref,
             bmm_ref, pm6_ref):
    ip = jnp.where(m3_ref[0] > 0.5, o3_ref[0],
                   jnp.where(m2_ref[0] > 0.5, o2_ref[0],
                             jnp.where(m1_ref[0] > 0.5, o1_ref[0],
                                       jnp.where(m0_ref[0] > 0.5, o0_ref[0],
                                                 p1_ref[0]))))
    add = ip + p7_ref[0] * 0.975
    bmm_ref[0] = jnp.dot(v_ref[0], add, preferred_element_type=jnp.float32)
    pm6_ref[0] = add.T


_b3 = pl.BlockSpec((1, 256, 256), lambda b: (b, 0, 0))
_tc_call = pl.pallas_call(
    _tc_body,
    grid=(6,),
    in_specs=[
        _b3, _b3, _b3, _b3, _b3, _b3, _b3, _b3, _b3, _b3,
        pl.BlockSpec((1, 12, 256), lambda b: (b, 0, 0)),
    ],
    out_specs=[
        pl.BlockSpec((1, 12, 256), lambda b: (b, 0, 0)),
        _b3,
    ],
    out_shape=[
        jax.ShapeDtypeStruct((6, 12, 256), jnp.float32),
        jax.ShapeDtypeStruct((6, 256, 256), jnp.float32),
    ],
)


@jax.jit
def kernel(primals_1, primals_2, primals_3, primals_4, primals_5, primals_6,
           primals_7, primals_8):
    p2lin, dest, lin = _prep_call(primals_2, primals_3, primals_4,
                                  primals_5, primals_6)
    vals = _gather_call(p2lin, lin)
    so = _scatter_call(dest, vals)
    cubes = [a.reshape(6, 256, 256) for a in so]
    view = jnp.transpose(primals_8, (1, 0, 2))
    bmm6, pm6 = _tc_call(*cubes, primals_1, primals_7, view)
    view_3 = jnp.transpose(bmm6, (1, 0, 2))
    return (view_3, pm6)
